# Initial kernel scaffold; baseline (speedup 1.0000x reference)
import jax
import jax.numpy as jnp
from jax.experimental import pallas as pl


def kernel(diffusion_step, embedding, W1, b1, W2, b2):
    raise NotImplementedError("write your pallas kernel here")


# speedup vs baseline: 1.2187x; 1.2187x over previous
import functools

import jax
import jax.numpy as jnp
from jax import lax
from jax.experimental import pallas as pl
from jax.experimental.pallas import tpu as pltpu
from jax.experimental.pallas import tpu_sc as plsc

_BATCH = 16384
_ROWS = 1000
_D_IN = 64
_D_OUT = 64


def _mlp_body(emb_ref, w1_ref, b1_ref, w2_ref, b2_ref, out_ref):
    x = emb_ref[...]
    h = jnp.dot(x, w1_ref[...], preferred_element_type=jnp.float32) + b1_ref[...]
    h = h * jnp.tanh(jax.nn.softplus(h))
    out_ref[...] = (
        jnp.dot(h, w2_ref[...], preferred_element_type=jnp.float32) + b2_ref[...]
    )


def _table_mlp(embedding, W1, b1, W2, b2):
    return pl.pallas_call(
        _mlp_body,
        out_shape=jax.ShapeDtypeStruct((_ROWS, _D_OUT), jnp.float32),
    )(embedding, W1, b1.reshape(1, -1), W2, b2.reshape(1, -1))


def _make_sc_gather():
    info = plsc.get_sparse_core_info()
    nw = info.num_cores * info.num_subcores
    bpw = _BATCH // nw
    ch = 128
    nch = bpw // ch
    mesh = plsc.VectorSubcoreMesh(core_axis_name="c", subcore_axis_name="s")

    @functools.partial(
        pl.kernel,
        mesh=mesh,
        compiler_params=pltpu.CompilerParams(use_tc_tiling_on_sc=False),
        out_type=jax.ShapeDtypeStruct((_BATCH, _D_OUT), jnp.float32),
        scratch_types=[
            pltpu.VMEM((nch, ch), jnp.int32),
            pltpu.VMEM((bpw, _D_OUT), jnp.float32),
            pltpu.SemaphoreType.DMA,
        ],
    )
    def gather(tbl_hbm, idx_hbm, out_hbm, idx_v, rows_v, sem):
        wid = lax.axis_index("s") * info.num_cores + lax.axis_index("c")
        base = wid * bpw
        pltpu.sync_copy(idx_hbm.at[pl.ds(wid * nch, nch)], idx_v)
        copies = [
            pltpu.async_copy(
                tbl_hbm.at[idx_v.at[j]], rows_v.at[pl.ds(j * ch, ch)], sem
            )
            for j in range(nch)
        ]
        for c in copies:
            c.wait()
        pltpu.sync_copy(rows_v, out_hbm.at[pl.ds(base, bpw)])

    return gather


_sc_gather = _make_sc_gather()


def kernel(diffusion_step, embedding, W1, b1, W2, b2):
    tbl_out = _table_mlp(embedding, W1, b1, W2, b2)
    idx = diffusion_step.astype(jnp.int32).reshape(-1, 128)
    return _sc_gather(tbl_out, idx)
